# baseline (speedup 1.0000x reference)
import functools

import jax
import jax.numpy as jnp
from jax import lax
from jax.experimental import pallas as pl
from jax.experimental.pallas import tpu as pltpu
from jax.experimental.pallas import tpu_sc as plsc

_LANES = 16
_CHUNK = 512
_SUB = 128
_BLK = 512


def _transpose_tables(tables_t, d, r, info):
    num_workers = info.num_cores * info.num_subcores
    nfull = d // _BLK
    tail = d % _BLK
    base_blocks = nfull // num_workers
    extra = nfull % num_workers

    mesh = plsc.VectorSubcoreMesh(core_axis_name="c", subcore_axis_name="s")

    @functools.partial(
        pl.kernel,
        out_type=jax.ShapeDtypeStruct((3 * d * r,), jnp.float32),
        mesh=mesh,
        scratch_types=[
            [pltpu.VMEM((r, _BLK), jnp.float32)] * 2,
            [pltpu.VMEM((_BLK * r,), jnp.float32)] * 2,
            pltpu.VMEM((r, tail), jnp.float32) if tail else None,
            pltpu.VMEM((tail * r,), jnp.float32) if tail else None,
            [pltpu.SemaphoreType.DMA] * 2,
            [pltpu.SemaphoreType.DMA] * 2,
        ],
        compiler_params=pltpu.CompilerParams(
            needs_layout_passes=False, use_tc_tiling_on_sc=True
        ),
    )
    def tbody(t0, t1, t2, out_hbm, vbuf, obuf, vtail, otail, isem, osem):
        wid = lax.axis_index("s") * info.num_cores + lax.axis_index("c")
        iota = lax.iota(jnp.int32, _LANES)
        tables = (t0, t1, t2)
        cvecs = [(iota + dd) & (r - 1) for dd in range(r)]
        svecs = [iota * r + cv for cv in cvecs]

        def transpose_block(vb, ob, nrows):
            def do_group(g, carry):
                i0 = g * _LANES
                rows = i0 + iota
                for dd in range(r):
                    v = plsc.load_gather(vb, [cvecs[dd], rows])
                    plsc.store_scatter(ob, [svecs[dd] + i0 * r], v)
                return carry

            lax.fori_loop(0, nrows // _LANES, do_group, 0)

        def run_table(t):
            tbl = tables[t]
            nb = base_blocks + jnp.where(wid < extra, 1, 0)

            def fire(k, slot):
                b = wid + k * num_workers
                pltpu.async_copy(
                    tbl.at[:, pl.ds(b * _BLK, _BLK)], vbuf[slot], isem[slot]
                )

            def wait_in(slot):
                pltpu.make_async_copy(
                    tbl.at[:, pl.ds(0, _BLK)], vbuf[slot], isem[slot]
                ).wait()

            def flush(k, slot):
                b = wid + k * num_workers
                pltpu.async_copy(
                    obuf[slot],
                    out_hbm.at[pl.ds((t * d + b * _BLK) * r, _BLK * r)],
                    osem[slot],
                )

            def drain_out(slot):
                pltpu.make_async_copy(
                    out_hbm.at[pl.ds(0, _BLK * r)], obuf[slot], osem[slot]
                ).wait()

            def stage(x, slot):
                @pl.when(x < nb)
                def _():
                    @pl.when(x + 1 < nb)
                    def _():
                        fire(x + 1, 1 - slot)

                    wait_in(slot)

                    @pl.when(x >= 2)
                    def _():
                        drain_out(slot)

                    transpose_block(vbuf[slot], obuf[slot], _BLK)
                    flush(x, slot)

            fire(0, 0)
            nb_max = base_blocks + (1 if extra else 0)

            def pair_body(k2, carry):
                a = 2 * k2
                stage(a, 0)
                stage(a + 1, 1)
                return carry

            lax.fori_loop(0, (nb_max + 1) // 2, pair_body, 0)
            drain_out(0)

            @pl.when(nb >= 2)
            def _():
                drain_out(1)

        for t in range(3):
            run_table(t)

        if tail:
            for t in range(3):

                @pl.when(wid == 5 + t)
                def _():
                    tbl = tables[t]
                    pltpu.sync_copy(tbl.at[:, pl.ds(nfull * _BLK, tail)], vtail)
                    transpose_block(vtail, otail, tail)
                    pltpu.sync_copy(
                        otail,
                        out_hbm.at[pl.ds((t * d + nfull * _BLK) * r, tail * r)],
                    )

    return tbody(*tables_t)


def _gather_reduce(fcat, i0x, i1x, i2x, n, r, info):
    num_workers = info.num_cores * info.num_subcores
    n_per_w = n // num_workers
    n_chunks = n_per_w // _CHUNK

    mesh = plsc.VectorSubcoreMesh(core_axis_name="c", subcore_axis_name="s")

    idx_t = pltpu.VMEM((_CHUNK,), jnp.int32)
    gbuf_t = pltpu.VMEM((_CHUNK, r), jnp.float32)
    out_t = pltpu.VMEM((_CHUNK,), jnp.float32)

    @functools.partial(
        pl.kernel,
        out_type=jax.ShapeDtypeStruct((n,), jnp.float32),
        mesh=mesh,
        scratch_types=[
            [idx_t] * 3,
            [idx_t] * 3,
            [gbuf_t] * 3,
            [gbuf_t] * 3,
            [out_t] * 2,
            pltpu.SemaphoreType.DMA,
            pltpu.SemaphoreType.DMA,
            pltpu.SemaphoreType.DMA,
            pltpu.SemaphoreType.DMA,
        ],
        compiler_params=pltpu.CompilerParams(
            needs_layout_passes=False, use_tc_tiling_on_sc=False
        ),
    )
    def body(fcat_hbm, i0_hbm, i1_hbm, i2_hbm, out_hbm,
             idx_a, idx_b, g_a, g_b, ov, sem_a, sem_b, osem_a, osem_b):
        wid = lax.axis_index("s") * info.num_cores + lax.axis_index("c")
        wbase = wid * n_per_w
        iota = lax.iota(jnp.int32, _LANES)
        idx_hbms = (i0_hbm, i1_hbm, i2_hbm)

        def fire(c, idxs, gs, sem):
            base = wbase + c * _CHUNK
            for t in range(3):
                pltpu.sync_copy(idx_hbms[t].at[pl.ds(base, _CHUNK)], idxs[t])
            for t in range(3):
                for j in range(_CHUNK // _SUB):
                    s = pl.ds(j * _SUB, _SUB)
                    pltpu.async_copy(fcat_hbm.at[idxs[t].at[s]], gs[t].at[s], sem)

        def drain(gs, sem):
            for t in range(3):
                pltpu.make_async_copy(
                    fcat_hbm.at[pl.ds(0, _CHUNK)], gs[t], sem
                ).wait()

        def compute(c, gs, o, osem):
            g0, g1, g2 = gs

            def do_group(g, gcarry):
                rows = g * _LANES + iota
                acc = jnp.zeros((_LANES,), jnp.float32)
                for rr in range(r):
                    col = (iota + rr) & (r - 1)
                    v0 = plsc.load_gather(g0, [rows, col])
                    v1 = plsc.load_gather(g1, [rows, col])
                    v2 = plsc.load_gather(g2, [rows, col])
                    acc = acc + v0 * v1 * v2
                o[pl.ds(g * _LANES, _LANES)] = acc
                return gcarry

            lax.fori_loop(0, _CHUNK // _LANES, do_group, 0)
            base = wbase + c * _CHUNK
            pltpu.async_copy(o, out_hbm.at[pl.ds(base, _CHUNK)], osem)

        def owait(o, osem):
            pltpu.make_async_copy(out_hbm.at[pl.ds(0, _CHUNK)], o, osem).wait()

        fire(0, idx_a, g_a, sem_a)

        def pair_body(k, carry):
            a = 2 * k
            b = a + 1
            fire(b, idx_b, g_b, sem_b)
            drain(g_a, sem_a)

            @pl.when(k > 0)
            def _():
                owait(ov[0], osem_a)

            compute(a, g_a, ov[0], osem_a)

            @pl.when(b + 1 < n_chunks)
            def _():
                fire(b + 1, idx_a, g_a, sem_a)

            drain(g_b, sem_b)

            @pl.when(k > 0)
            def _():
                owait(ov[1], osem_b)

            compute(b, g_b, ov[1], osem_b)
            return carry

        lax.fori_loop(0, n_chunks // 2, pair_body, 0)
        owait(ov[0], osem_a)
        owait(ov[1], osem_b)

    return body(fcat, i0x, i1x, i2x)


def kernel(F0, F1, F2, indices_list):
    n = indices_list.shape[1]
    d, r = F0.shape
    info = plsc.get_sparse_core_info()

    flat = _transpose_tables((F0.T, F1.T, F2.T), d, r, info)
    fcat = flat.reshape(3 * d, r)
    return _gather_reduce(
        fcat,
        indices_list[0],
        indices_list[1] + d,
        indices_list[2] + 2 * d,
        n,
        r,
        info,
    )


# speedup vs baseline: 1.0932x; 1.0932x over previous
import functools

import jax
import jax.numpy as jnp
from jax import lax
from jax.experimental import pallas as pl
from jax.experimental.pallas import tpu as pltpu
from jax.experimental.pallas import tpu_sc as plsc

_LANES = 16
_CHUNK = 512
_SUB = 128
_BLK = 512


def _transpose_tables(tables_t, d, r, info):
    num_workers = info.num_cores * info.num_subcores
    nfull = d // _BLK
    tail = d % _BLK
    base_blocks = nfull // num_workers
    extra = nfull % num_workers

    mesh = plsc.VectorSubcoreMesh(core_axis_name="c", subcore_axis_name="s")

    @functools.partial(
        pl.kernel,
        out_type=jax.ShapeDtypeStruct((3 * d * r,), jnp.float32),
        mesh=mesh,
        scratch_types=[
            [pltpu.VMEM((r, _BLK), jnp.float32)] * 2,
            [pltpu.VMEM((_BLK * r,), jnp.float32)] * 2,
            pltpu.VMEM((r, tail), jnp.float32) if tail else None,
            pltpu.VMEM((tail * r,), jnp.float32) if tail else None,
            [pltpu.SemaphoreType.DMA] * 2,
            [pltpu.SemaphoreType.DMA] * 2,
        ],
        compiler_params=pltpu.CompilerParams(
            needs_layout_passes=False, use_tc_tiling_on_sc=True
        ),
    )
    def tbody(t0, t1, t2, out_hbm, vbuf, obuf, vtail, otail, isem, osem):
        wid = lax.axis_index("s") * info.num_cores + lax.axis_index("c")
        iota = lax.iota(jnp.int32, _LANES)
        tables = (t0, t1, t2)
        cvecs = [(iota + dd) & (r - 1) for dd in range(r)]
        svecs = [iota * r + cv for cv in cvecs]

        def transpose_block(vb, ob, nrows):
            def do_group(g, carry):
                i0 = g * _LANES
                rows = i0 + iota
                obs = ob.at[pl.ds(i0 * r, _LANES * r)]
                for dd in range(r):
                    v = plsc.load_gather(vb, [cvecs[dd], rows])
                    plsc.store_scatter(obs, [svecs[dd]], v)
                return carry

            lax.fori_loop(0, nrows // _LANES, do_group, 0)

        def run_table(t):
            tbl = tables[t]
            nb = base_blocks + jnp.where(wid < extra, 1, 0)

            def fire(k, slot):
                b = wid + k * num_workers
                pltpu.async_copy(
                    tbl.at[:, pl.ds(b * _BLK, _BLK)], vbuf[slot], isem[slot]
                )

            def wait_in(slot):
                pltpu.make_async_copy(
                    tbl.at[:, pl.ds(0, _BLK)], vbuf[slot], isem[slot]
                ).wait()

            def flush(k, slot):
                b = wid + k * num_workers
                pltpu.async_copy(
                    obuf[slot],
                    out_hbm.at[pl.ds((t * d + b * _BLK) * r, _BLK * r)],
                    osem[slot],
                )

            def drain_out(slot):
                pltpu.make_async_copy(
                    out_hbm.at[pl.ds(0, _BLK * r)], obuf[slot], osem[slot]
                ).wait()

            def stage(x, slot):
                @pl.when(x < nb)
                def _():
                    @pl.when(x + 1 < nb)
                    def _():
                        fire(x + 1, 1 - slot)

                    wait_in(slot)

                    @pl.when(x >= 2)
                    def _():
                        drain_out(slot)

                    transpose_block(vbuf[slot], obuf[slot], _BLK)
                    flush(x, slot)

            fire(0, 0)
            nb_max = base_blocks + (1 if extra else 0)

            def pair_body(k2, carry):
                a = 2 * k2
                stage(a, 0)
                stage(a + 1, 1)
                return carry

            lax.fori_loop(0, (nb_max + 1) // 2, pair_body, 0)
            drain_out(0)

            @pl.when(nb >= 2)
            def _():
                drain_out(1)

        for t in range(3):
            run_table(t)

        if tail:
            for t in range(3):

                @pl.when(wid == 5 + t)
                def _():
                    tbl = tables[t]
                    pltpu.sync_copy(tbl.at[:, pl.ds(nfull * _BLK, tail)], vtail)
                    transpose_block(vtail, otail, tail)
                    pltpu.sync_copy(
                        otail,
                        out_hbm.at[pl.ds((t * d + nfull * _BLK) * r, tail * r)],
                    )

    return tbody(*tables_t)


def _gather_reduce(fcat, i0x, i1x, i2x, n, r, info):
    num_workers = info.num_cores * info.num_subcores
    n_per_w = n // num_workers
    n_chunks = n_per_w // _CHUNK

    mesh = plsc.VectorSubcoreMesh(core_axis_name="c", subcore_axis_name="s")

    idx_t = pltpu.VMEM((_CHUNK,), jnp.int32)
    gbuf_t = pltpu.VMEM((_CHUNK, r), jnp.float32)
    out_t = pltpu.VMEM((_CHUNK,), jnp.float32)

    @functools.partial(
        pl.kernel,
        out_type=jax.ShapeDtypeStruct((n,), jnp.float32),
        mesh=mesh,
        scratch_types=[
            [idx_t] * 3,
            [idx_t] * 3,
            [gbuf_t] * 3,
            [gbuf_t] * 3,
            [out_t] * 2,
            pltpu.SemaphoreType.DMA,
            pltpu.SemaphoreType.DMA,
            pltpu.SemaphoreType.DMA,
            pltpu.SemaphoreType.DMA,
        ],
        compiler_params=pltpu.CompilerParams(
            needs_layout_passes=False, use_tc_tiling_on_sc=False
        ),
    )
    def body(fcat_hbm, i0_hbm, i1_hbm, i2_hbm, out_hbm,
             idx_a, idx_b, g_a, g_b, ov, sem_a, sem_b, osem_a, osem_b):
        wid = lax.axis_index("s") * info.num_cores + lax.axis_index("c")
        wbase = wid * n_per_w
        iota = lax.iota(jnp.int32, _LANES)
        idx_hbms = (i0_hbm, i1_hbm, i2_hbm)

        def fire(c, idxs, gs, sem):
            base = wbase + c * _CHUNK
            for t in range(3):
                pltpu.sync_copy(idx_hbms[t].at[pl.ds(base, _CHUNK)], idxs[t])
            for t in range(3):
                for j in range(_CHUNK // _SUB):
                    s = pl.ds(j * _SUB, _SUB)
                    pltpu.async_copy(fcat_hbm.at[idxs[t].at[s]], gs[t].at[s], sem)

        def drain(gs, sem):
            for t in range(3):
                pltpu.make_async_copy(
                    fcat_hbm.at[pl.ds(0, _CHUNK)], gs[t], sem
                ).wait()

        cols = [(iota + rr) & (r - 1) for rr in range(r)]

        def compute(c, gs, o, osem):
            g0, g1, g2 = gs

            def do_group(g, gcarry):
                s = pl.ds(g * _LANES, _LANES)
                g0s, g1s, g2s = g0.at[s], g1.at[s], g2.at[s]
                acc = jnp.zeros((_LANES,), jnp.float32)
                for rr in range(r):
                    v0 = plsc.load_gather(g0s, [iota, cols[rr]])
                    v1 = plsc.load_gather(g1s, [iota, cols[rr]])
                    v2 = plsc.load_gather(g2s, [iota, cols[rr]])
                    acc = acc + v0 * v1 * v2
                o[s] = acc
                return gcarry

            lax.fori_loop(0, _CHUNK // _LANES, do_group, 0)
            base = wbase + c * _CHUNK
            pltpu.async_copy(o, out_hbm.at[pl.ds(base, _CHUNK)], osem)

        def owait(o, osem):
            pltpu.make_async_copy(out_hbm.at[pl.ds(0, _CHUNK)], o, osem).wait()

        fire(0, idx_a, g_a, sem_a)

        def pair_body(k, carry):
            a = 2 * k
            b = a + 1
            fire(b, idx_b, g_b, sem_b)
            drain(g_a, sem_a)

            @pl.when(k > 0)
            def _():
                owait(ov[0], osem_a)

            compute(a, g_a, ov[0], osem_a)

            @pl.when(b + 1 < n_chunks)
            def _():
                fire(b + 1, idx_a, g_a, sem_a)

            drain(g_b, sem_b)

            @pl.when(k > 0)
            def _():
                owait(ov[1], osem_b)

            compute(b, g_b, ov[1], osem_b)
            return carry

        lax.fori_loop(0, n_chunks // 2, pair_body, 0)
        owait(ov[0], osem_a)
        owait(ov[1], osem_b)

    return body(fcat, i0x, i1x, i2x)


def kernel(F0, F1, F2, indices_list):
    n = indices_list.shape[1]
    d, r = F0.shape
    info = plsc.get_sparse_core_info()

    flat = _transpose_tables((F0.T, F1.T, F2.T), d, r, info)
    fcat = flat.reshape(3 * d, r)
    return _gather_reduce(
        fcat,
        indices_list[0],
        indices_list[1] + d,
        indices_list[2] + 2 * d,
        n,
        r,
        info,
    )


# speedup vs baseline: 1.1542x; 1.0558x over previous
import functools

import jax
import jax.numpy as jnp
from jax import lax
from jax.experimental import pallas as pl
from jax.experimental.pallas import tpu as pltpu
from jax.experimental.pallas import tpu_sc as plsc

_LANES = 16
_CHUNK = 512
_SUB = 128
_BLK = 512


def _transpose_tables(tables_t, d, r, info):
    num_workers = info.num_cores * info.num_subcores
    nt = len(tables_t)
    nfull = d // _BLK
    tail = d % _BLK
    base_blocks = nfull // num_workers
    extra = nfull % num_workers

    mesh = plsc.VectorSubcoreMesh(core_axis_name="c", subcore_axis_name="s")

    @functools.partial(
        pl.kernel,
        out_type=jax.ShapeDtypeStruct((nt * d * r,), jnp.float32),
        mesh=mesh,
        scratch_types=[
            [pltpu.VMEM((r, _BLK), jnp.float32)] * 2,
            [pltpu.VMEM((_BLK * r,), jnp.float32)] * 2,
            pltpu.VMEM((r, tail), jnp.float32) if tail else None,
            pltpu.VMEM((tail * r,), jnp.float32) if tail else None,
            [pltpu.SemaphoreType.DMA] * 2,
            [pltpu.SemaphoreType.DMA] * 2,
        ],
        compiler_params=pltpu.CompilerParams(
            needs_layout_passes=False, use_tc_tiling_on_sc=True
        ),
    )
    def tbody(*args):
        tables = args[:nt]
        out_hbm, vbuf, obuf, vtail, otail, isem, osem = args[nt:]
        wid = lax.axis_index("s") * info.num_cores + lax.axis_index("c")
        iota = lax.iota(jnp.int32, _LANES)
        cvecs = [(iota + dd) & (r - 1) for dd in range(r)]
        svecs = [iota * r + cv for cv in cvecs]

        def transpose_block(vb, ob, nrows):
            def do_group(g, carry):
                i0 = g * _LANES
                rows = i0 + iota
                obs = ob.at[pl.ds(i0 * r, _LANES * r)]
                for dd in range(r):
                    v = plsc.load_gather(vb, [cvecs[dd], rows])
                    plsc.store_scatter(obs, [svecs[dd]], v)
                return carry

            lax.fori_loop(0, nrows // _LANES, do_group, 0)

        def run_table(t):
            tbl = tables[t]
            nb = base_blocks + jnp.where(wid < extra, 1, 0)

            def fire(k, slot):
                b = wid + k * num_workers
                pltpu.async_copy(
                    tbl.at[:, pl.ds(b * _BLK, _BLK)], vbuf[slot], isem[slot]
                )

            def wait_in(slot):
                pltpu.make_async_copy(
                    tbl.at[:, pl.ds(0, _BLK)], vbuf[slot], isem[slot]
                ).wait()

            def flush(k, slot):
                b = wid + k * num_workers
                pltpu.async_copy(
                    obuf[slot],
                    out_hbm.at[pl.ds((t * d + b * _BLK) * r, _BLK * r)],
                    osem[slot],
                )

            def drain_out(slot):
                pltpu.make_async_copy(
                    out_hbm.at[pl.ds(0, _BLK * r)], obuf[slot], osem[slot]
                ).wait()

            def stage(x, slot):
                @pl.when(x < nb)
                def _():
                    @pl.when(x + 1 < nb)
                    def _():
                        fire(x + 1, 1 - slot)

                    wait_in(slot)

                    @pl.when(x >= 2)
                    def _():
                        drain_out(slot)

                    transpose_block(vbuf[slot], obuf[slot], _BLK)
                    flush(x, slot)

            fire(0, 0)
            nb_max = base_blocks + (1 if extra else 0)

            def pair_body(k2, carry):
                a = 2 * k2
                stage(a, 0)
                stage(a + 1, 1)
                return carry

            lax.fori_loop(0, (nb_max + 1) // 2, pair_body, 0)
            drain_out(0)

            @pl.when(nb >= 2)
            def _():
                drain_out(1)

        for t in range(nt):
            run_table(t)

        if tail:
            for t in range(nt):

                @pl.when(wid == 5 + t)
                def _():
                    tbl = tables[t]
                    pltpu.sync_copy(tbl.at[:, pl.ds(nfull * _BLK, tail)], vtail)
                    transpose_block(vtail, otail, tail)
                    pltpu.sync_copy(
                        otail,
                        out_hbm.at[pl.ds((t * d + nfull * _BLK) * r, tail * r)],
                    )

    return tbody(*tables_t)


def _gather_reduce(fab, fc, i0x, i1x, i2x, n, r, info):
    num_workers = info.num_cores * info.num_subcores
    n_per_w = n // num_workers
    n_chunks = n_per_w // _CHUNK

    mesh = plsc.VectorSubcoreMesh(core_axis_name="c", subcore_axis_name="s")

    idx_t = pltpu.VMEM((_CHUNK,), jnp.int32)
    gbuf_t = pltpu.VMEM((_CHUNK, r), jnp.float32)
    out_t = pltpu.VMEM((_CHUNK,), jnp.float32)

    @functools.partial(
        pl.kernel,
        out_type=jax.ShapeDtypeStruct((n,), jnp.float32),
        mesh=mesh,
        scratch_types=[
            [idx_t] * 3,
            [idx_t] * 3,
            [gbuf_t] * 3,
            [gbuf_t] * 3,
            [out_t] * 2,
            pltpu.SemaphoreType.DMA,
            pltpu.SemaphoreType.DMA,
            pltpu.SemaphoreType.DMA,
            pltpu.SemaphoreType.DMA,
        ],
        compiler_params=pltpu.CompilerParams(
            needs_layout_passes=False, use_tc_tiling_on_sc=False
        ),
    )
    def body(fab_hbm, fc_hbm, i0_hbm, i1_hbm, i2_hbm, out_hbm,
             idx_a, idx_b, g_a, g_b, ov, sem_a, sem_b, osem_a, osem_b):
        wid = lax.axis_index("s") * info.num_cores + lax.axis_index("c")
        wbase = wid * n_per_w
        iota = lax.iota(jnp.int32, _LANES)
        idx_hbms = (i0_hbm, i1_hbm, i2_hbm)
        tbl_hbms = (fab_hbm, fab_hbm, fc_hbm)

        def fire(c, idxs, gs, sem):
            base = wbase + c * _CHUNK
            for t in range(3):
                pltpu.sync_copy(idx_hbms[t].at[pl.ds(base, _CHUNK)], idxs[t])
            for t in range(3):
                for j in range(_CHUNK // _SUB):
                    s = pl.ds(j * _SUB, _SUB)
                    pltpu.async_copy(
                        tbl_hbms[t].at[idxs[t].at[s]], gs[t].at[s], sem
                    )

        def drain(gs, sem):
            for t in range(3):
                pltpu.make_async_copy(
                    tbl_hbms[t].at[pl.ds(0, _CHUNK)], gs[t], sem
                ).wait()

        cols = [(iota + rr) & (r - 1) for rr in range(r)]

        def compute(c, gs, o, osem):
            g0, g1, g2 = gs

            def do_group(g, gcarry):
                s = pl.ds(g * _LANES, _LANES)
                g0s, g1s, g2s = g0.at[s], g1.at[s], g2.at[s]
                acc = jnp.zeros((_LANES,), jnp.float32)
                for rr in range(r):
                    v0 = plsc.load_gather(g0s, [iota, cols[rr]])
                    v1 = plsc.load_gather(g1s, [iota, cols[rr]])
                    v2 = plsc.load_gather(g2s, [iota, cols[rr]])
                    acc = acc + v0 * v1 * v2
                o[s] = acc
                return gcarry

            lax.fori_loop(0, _CHUNK // _LANES, do_group, 0)
            base = wbase + c * _CHUNK
            pltpu.async_copy(o, out_hbm.at[pl.ds(base, _CHUNK)], osem)

        def owait(o, osem):
            pltpu.make_async_copy(out_hbm.at[pl.ds(0, _CHUNK)], o, osem).wait()

        fire(0, idx_a, g_a, sem_a)

        def pair_body(k, carry):
            a = 2 * k
            b = a + 1
            fire(b, idx_b, g_b, sem_b)
            drain(g_a, sem_a)

            @pl.when(k > 0)
            def _():
                owait(ov[0], osem_a)

            compute(a, g_a, ov[0], osem_a)

            @pl.when(b + 1 < n_chunks)
            def _():
                fire(b + 1, idx_a, g_a, sem_a)

            drain(g_b, sem_b)

            @pl.when(k > 0)
            def _():
                owait(ov[1], osem_b)

            compute(b, g_b, ov[1], osem_b)
            return carry

        lax.fori_loop(0, n_chunks // 2, pair_body, 0)
        owait(ov[0], osem_a)
        owait(ov[1], osem_b)

    return body(fab, fc, i0x, i1x, i2x)


def kernel(F0, F1, F2, indices_list):
    n = indices_list.shape[1]
    d, r = F0.shape
    info = plsc.get_sparse_core_info()

    flat_ab = _transpose_tables((F0.T, F1.T), d, r, info)
    fab = flat_ab.reshape(2 * d, r)
    fc = F2.reshape(-1).reshape(d, r)
    return _gather_reduce(
        fab,
        fc,
        indices_list[0],
        indices_list[1] + d,
        indices_list[2],
        n,
        r,
        info,
    )
